# baseline (speedup 1.0000x reference)
import functools

import jax
import jax.numpy as jnp
import numpy as np
from jax import lax
from jax.experimental import pallas as pl
from jax.experimental.pallas import tpu as pltpu
from jax.experimental.pallas import tpu_sc as plsc


def _add_body(m_ref, r_ref, o_ref):
    o_ref[...] = m_ref[...] + r_ref[...]


def _combined_feat(memory_state, node_raw_features):
    n, d = memory_state.shape
    blk = 2000
    assert n % blk == 0
    return pl.pallas_call(
        _add_body,
        grid=(n // blk,),
        in_specs=[pl.BlockSpec((blk, d), lambda i: (i, 0)),
                  pl.BlockSpec((blk, d), lambda i: (i, 0))],
        out_specs=pl.BlockSpec((blk, d), lambda i: (i, 0)),
        out_shape=jax.ShapeDtypeStruct((n, d), jnp.float32),
    )(memory_state, node_raw_features)


_SEG_ROWS = 131072


def _repack_body(*refs):
    eye_ref = refs[-2]
    out_ref = refs[-1]
    de = refs[0].shape[1]
    acc = None
    for s, r in enumerate(refs[:-2]):
        part = lax.dot_general(r[...], eye_ref[s * de:(s + 1) * de, :],
                               (((1,), (0,)), ((), ())),
                               preferred_element_type=jnp.float32)
        acc = part if acc is None else acc + part
    out_ref[...] = acc


def _repack_edges(edge_raw):
    e_cnt, de = edge_raw.shape
    pack = 128 // de
    n_seg = -(-e_cnt // _SEG_ROWS)
    assert n_seg <= pack
    total = n_seg * _SEG_ROWS
    pad = jnp.zeros((total - e_cnt, de), edge_raw.dtype)
    er_pad = jnp.concatenate([edge_raw, pad], axis=0)
    blk = 4096
    nblk = _SEG_ROWS // blk
    in_specs = [pl.BlockSpec((blk, de), (lambda i, s=s: (s * nblk + i, 0)))
                for s in range(n_seg)]
    w = pack * de
    in_specs.append(pl.BlockSpec((w, w), lambda i: (0, 0)))
    return pl.pallas_call(
        _repack_body,
        grid=(nblk,),
        in_specs=in_specs,
        out_specs=pl.BlockSpec((blk, w), lambda i: (i, 0)),
        out_shape=jax.ShapeDtypeStruct((_SEG_ROWS, w), jnp.float32),
    )(*([er_pad] * n_seg + [jnp.eye(w, dtype=jnp.float32)]))


def _sc_gather_all(nodes, neighbors, ngh_e_tab, ngh_t_tab, feat, er_big):
    b = nodes.shape[0]
    info = plsc.get_sparse_core_info()
    nc, ns = info.num_cores, info.num_subcores
    nw = nc * ns
    assert b % nw == 0
    bw = b // nw
    k = neighbors.shape[1]
    d = feat.shape[1]
    dbig = er_big.shape[1]
    ch = k
    nch = bw
    nbuf = 6
    look = 4
    assert nch % nbuf == 0
    assert k % 16 == 0
    mesh = plsc.VectorSubcoreMesh(core_axis_name="c", subcore_axis_name="s")
    f32 = jnp.float32
    i32 = jnp.int32

    @functools.partial(
        pl.kernel,
        mesh=mesh,
        out_type=(jax.ShapeDtypeStruct((b, k), f32),
                  jax.ShapeDtypeStruct((b, k), i32),
                  jax.ShapeDtypeStruct((b, d), f32),
                  jax.ShapeDtypeStruct((b * k, d), f32),
                  jax.ShapeDtypeStruct((b * k, dbig), f32)),
        scratch_types=[pltpu.VMEM((bw,), i32),
                       pltpu.VMEM((bw, k), i32),
                       pltpu.VMEM((bw, k), i32),
                       pltpu.VMEM((bw, k), i32),
                       pltpu.VMEM((bw, k), f32),
                       pltpu.VMEM((bw, d), f32)]
                      + [pltpu.VMEM((ch, d), f32)] * nbuf
                      + [pltpu.VMEM((ch, dbig), f32)] * nbuf
                      + [pltpu.SemaphoreType.DMA] * (2 * nbuf + 1),
        compiler_params=pltpu.CompilerParams(use_tc_tiling_on_sc=False),
    )
    def gather(nodes_h, ngh_h, nghe_h, nght_h, feat_h, er_h,
               nght_o, nghe_o, src_o, nf_o, ef_o,
               idx_v, n_v, e_v, es_v, t_v, s_v, *bufs):
        rows = bufs[0:nbuf]
        erows = bufs[nbuf:2 * nbuf]
        in_sem = bufs[2 * nbuf:3 * nbuf]
        out_sem = bufs[3 * nbuf:4 * nbuf]
        sem0 = bufs[4 * nbuf]
        wid = lax.axis_index("s") * nc + lax.axis_index("c")
        base = wid * bw

        pltpu.sync_copy(nodes_h.at[pl.ds(base, bw)], idx_v)
        c1 = pltpu.async_copy(ngh_h.at[idx_v], n_v, sem0)
        c2 = pltpu.async_copy(nghe_h.at[idx_v], e_v, sem0)
        c3 = pltpu.async_copy(nght_h.at[idx_v], t_v, sem0)
        c4 = pltpu.async_copy(feat_h.at[idx_v], s_v, sem0)
        c1.wait()
        c2.wait()
        c3.wait()
        c4.wait()

        def shift_row(r, carry):
            for j in range(k // 16):
                sl = pl.ds(j * 16, 16)
                es_v[r, sl] = lax.bitwise_and(e_v[r, sl], _SEG_ROWS - 1)
            return carry

        lax.fori_loop(0, bw, shift_row, 0)

        pltpu.sync_copy(t_v, nght_o.at[pl.ds(base, bw)])
        pltpu.sync_copy(e_v, nghe_o.at[pl.ds(base, bw)])
        pltpu.sync_copy(s_v, src_o.at[pl.ds(base, bw)])

        def fire_in(s, c):
            pltpu.async_copy(feat_h.at[n_v.at[c]], rows[s], in_sem[s])
            pltpu.async_copy(er_h.at[es_v.at[c]], erows[s], in_sem[s])

        def wait_in(s):
            pltpu.make_async_copy(feat_h.at[n_v.at[0]], rows[s],
                                  in_sem[s]).wait()
            pltpu.make_async_copy(er_h.at[es_v.at[0]], erows[s],
                                  in_sem[s]).wait()

        def fire_out(s, c):
            off = (base + c) * k
            pltpu.async_copy(rows[s], nf_o.at[pl.ds(off, ch)], out_sem[s])
            pltpu.async_copy(erows[s], ef_o.at[pl.ds(off, ch)], out_sem[s])

        def wait_out(s):
            pltpu.make_async_copy(rows[s], nf_o.at[pl.ds(0, ch)],
                                  out_sem[s]).wait()
            pltpu.make_async_copy(erows[s], ef_o.at[pl.ds(0, ch)],
                                  out_sem[s]).wait()

        for c0 in range(look):
            fire_in(c0, c0)

        def outer(g, carry):
            for s in range(nbuf):
                c = g * nbuf + s
                wait_in(s)
                fire_out(s, c)
                c2 = c + look
                s2 = (s + look) % nbuf

                @pl.when(c2 < nch)
                def _():
                    @pl.when(c2 >= nbuf)
                    def _():
                        wait_out(s2)
                    fire_in(s2, c2)
            return carry

        lax.fori_loop(0, nch // nbuf, outer, 0)
        for s in range(nbuf):
            wait_out(s)

    return gather(nodes, neighbors, ngh_e_tab, ngh_t_tab, feat, er_big)


_INV_2PI = 0.15915494309189535
_PI_HI = 6.28125
_PI_LO = 0.0019353071795864769


def _cos(x):
    f32 = jnp.float32
    kf = jnp.round(x * f32(_INV_2PI))
    r = x - kf * f32(_PI_HI)
    r = r - kf * f32(_PI_LO)
    z = r * r
    p = f32(-1.0 / 87178291200.0)
    for coef in (1.0 / 479001600.0, -1.0 / 3628800.0, 1.0 / 40320.0,
                 -1.0 / 720.0, 1.0 / 24.0, -0.5, 1.0):
        p = p * z + f32(coef)
    return p


def _dense_body(src_ref, nf_ref, ef_ref, nghe_ref, ts_ref, nt_ref,
                tw_ref, tb_ref,
                wqn_ref, wqt_ref, wkn_ref, wkeb_ref, wkt_ref,
                wvn_ref, wveb_ref, wvt_ref, wm1a_ref, wm1b_ref, wm2_ref,
                out_ref):
    bq, d = src_ref.shape
    k = nt_ref.shape[1]
    dh = d // 2
    f32 = jnp.float32

    def mm(a, b):
        return lax.dot_general(a, b, (((1,), (0,)), ((), ())),
                               preferred_element_type=f32)

    src = src_ref[...]
    tw = tw_ref[...]
    tb = tb_ref[...]
    delta = ts_ref[...] - nt_ref[...]
    t_enc = _cos(delta[:, :, None] * tw[None, :, :] + tb[None, :, :])
    t2 = t_enc.reshape(bq * k, d)

    seg = nghe_ref[...] // _SEG_ROWS
    lane = lax.broadcasted_iota(jnp.int32, (1, 1, d), 2) >> 4
    mask = (lane == seg[:, :, None]).astype(f32)
    ef3 = ef_ref[...].reshape(bq, k, d)
    efm = (ef3 * mask).reshape(bq * k, d)

    nf = nf_ref[...]
    kk = mm(nf, wkn_ref[...]) + mm(efm, wkeb_ref[...]) + mm(t2, wkt_ref[...])
    vv = mm(nf, wvn_ref[...]) + mm(efm, wveb_ref[...]) + mm(t2, wvt_ref[...])

    qt = _cos(tb)
    q = mm(src, wqn_ref[...]) + mm(qt, wqt_ref[...])

    k3 = kk.reshape(bq, k, d)
    prod = k3 * q[:, None, :]
    scale = f32(1.0 / np.sqrt(dh))
    s0 = jnp.sum(prod[:, :, :dh], axis=-1) * scale
    s1 = jnp.sum(prod[:, :, dh:], axis=-1) * scale

    def softmax(s):
        m = jnp.max(s, axis=-1, keepdims=True)
        e = jnp.exp(s - m)
        return e / jnp.sum(e, axis=-1, keepdims=True)

    a0 = softmax(s0)
    a1 = softmax(s1)
    v3 = vv.reshape(bq, k, d)
    o0 = jnp.sum(a0[:, :, None] * v3[:, :, :dh], axis=1)
    o1 = jnp.sum(a1[:, :, None] * v3[:, :, dh:], axis=1)
    out = jnp.concatenate([o0, o1], axis=-1)

    hmid = jnp.maximum(mm(out, wm1a_ref[...]) + mm(src, wm1b_ref[...]), 0.0)
    out_ref[...] = mm(hmid, wm2_ref[...])


def _dense(src_feat, nf, ef, nghe, ts2, ngh_t, tw2, tb2,
           wqn, wqt, wkn, wkeb, wkt, wvn, wveb, wvt, wm1a, wm1b, wm2):
    b3, d = src_feat.shape
    k = ngh_t.shape[1]
    bq = 128
    assert b3 % bq == 0
    grid = (b3 // bq,)
    full = lambda shape: pl.BlockSpec(shape, lambda i: tuple(0 for _ in shape))
    return pl.pallas_call(
        _dense_body,
        grid=grid,
        in_specs=[
            pl.BlockSpec((bq, d), lambda i: (i, 0)),
            pl.BlockSpec((bq * k, d), lambda i: (i, 0)),
            pl.BlockSpec((bq * k, d), lambda i: (i, 0)),
            pl.BlockSpec((bq, k), lambda i: (i, 0)),
            pl.BlockSpec((bq, 1), lambda i: (i, 0)),
            pl.BlockSpec((bq, k), lambda i: (i, 0)),
            full((1, d)), full((1, d)),
            full((d, d)), full((d, d)),
            full((d, d)), full((d, d)), full((d, d)),
            full((d, d)), full((d, d)), full((d, d)),
            full((d, d)), full((d, d)), full((d, d)),
        ],
        out_specs=pl.BlockSpec((bq, d), lambda i: (i, 0)),
        out_shape=jax.ShapeDtypeStruct((b3, d), jnp.float32),
    )(src_feat, nf, ef, nghe, ts2, ngh_t, tw2, tb2,
      wqn, wqt, wkn, wkeb, wkt, wvn, wveb, wvt, wm1a, wm1b, wm2)


def kernel(source_nodes, destination_nodes, negative_nodes, edge_times,
           edge_idxs, node_raw_features, edge_raw_features, memory_state,
           neighbors, neighbor_edge_idxs, neighbor_times,
           time_w, time_b, W_q, W_k, W_v, W_m1, W_m2):
    del edge_idxs
    d = node_raw_features.shape[1]
    de = edge_raw_features.shape[1]
    e_cnt = edge_raw_features.shape[0]
    pack = 128 // de
    assert e_cnt % pack == 0

    nodes = jnp.concatenate(
        [source_nodes, destination_nodes, negative_nodes]).astype(jnp.int32)
    ts = jnp.concatenate([edge_times, edge_times, edge_times])
    er_big = _repack_edges(edge_raw_features)

    feat = _combined_feat(memory_state, node_raw_features)

    ngh_t, ngh_e, src_feat, nf, ef = _sc_gather_all(
        nodes, neighbors, neighbor_edge_idxs, neighbor_times, feat, er_big)

    tw2 = time_w.reshape(1, d)
    tb2 = time_b.reshape(1, d)
    wqn, wqt = W_q[:d], W_q[d:]
    wkn, wke, wkt = W_k[:d], W_k[d:d + de], W_k[d + de:]
    wvn, wve, wvt = W_v[:d], W_v[d:d + de], W_v[d + de:]
    wkeb = jnp.tile(wke, (pack, 1))
    wveb = jnp.tile(wve, (pack, 1))
    wm1a, wm1b = W_m1[:d], W_m1[d:]

    return _dense(src_feat, nf, ef, ngh_e, ts.reshape(-1, 1), ngh_t, tw2, tb2,
                  wqn, wqt, wkn, wkeb, wkt, wvn, wveb, wvt, wm1a, wm1b, W_m2)


# speedup vs baseline: 1.0403x; 1.0403x over previous
import functools

import jax
import jax.numpy as jnp
import numpy as np
from jax import lax
from jax.experimental import pallas as pl
from jax.experimental.pallas import tpu as pltpu
from jax.experimental.pallas import tpu_sc as plsc


def _add_body(m_ref, r_ref, o_ref):
    o_ref[...] = m_ref[...] + r_ref[...]


def _combined_feat(memory_state, node_raw_features):
    n, d = memory_state.shape
    blk = 2000
    assert n % blk == 0
    return pl.pallas_call(
        _add_body,
        grid=(n // blk,),
        in_specs=[pl.BlockSpec((blk, d), lambda i: (i, 0)),
                  pl.BlockSpec((blk, d), lambda i: (i, 0))],
        out_specs=pl.BlockSpec((blk, d), lambda i: (i, 0)),
        out_shape=jax.ShapeDtypeStruct((n, d), jnp.float32),
    )(memory_state, node_raw_features)


_SEG_ROWS = 131072


def _repack_body(e_cnt, blk, *refs):
    eye_ref = refs[-2]
    out_ref = refs[-1]
    de = refs[0].shape[1]
    i = pl.program_id(0)
    acc = None
    for s, r in enumerate(refs[:-2]):
        x = r[...]
        seg_end = (s + 1) * _SEG_ROWS
        if seg_end > e_cnt:
            row = (s * _SEG_ROWS + i * blk
                   + lax.broadcasted_iota(jnp.int32, x.shape, 0))
            x = jnp.where(row < e_cnt, x, 0.0)
        part = lax.dot_general(x, eye_ref[s * de:(s + 1) * de, :],
                               (((1,), (0,)), ((), ())),
                               preferred_element_type=jnp.float32)
        acc = part if acc is None else acc + part
    out_ref[...] = acc


def _repack_edges(edge_raw):
    e_cnt, de = edge_raw.shape
    pack = 128 // de
    n_seg = -(-e_cnt // _SEG_ROWS)
    assert n_seg <= pack
    blk = 1024
    assert e_cnt % blk == 0
    nblk = _SEG_ROWS // blk
    in_specs = [pl.BlockSpec((blk, de), (lambda i, s=s: (s * nblk + i, 0)))
                for s in range(n_seg)]
    w = pack * de
    in_specs.append(pl.BlockSpec((w, w), lambda i: (0, 0)))
    return pl.pallas_call(
        functools.partial(_repack_body, e_cnt, blk),
        grid=(nblk,),
        in_specs=in_specs,
        out_specs=pl.BlockSpec((blk, w), lambda i: (i, 0)),
        out_shape=jax.ShapeDtypeStruct((_SEG_ROWS, w), jnp.float32),
    )(*([edge_raw] * n_seg + [jnp.eye(w, dtype=jnp.float32)]))


def _sc_gather_all(nodes, neighbors, ngh_e_tab, ngh_t_tab, feat, er_big):
    b = nodes.shape[0]
    info = plsc.get_sparse_core_info()
    nc, ns = info.num_cores, info.num_subcores
    nw = nc * ns
    assert b % nw == 0
    bw = b // nw
    k = neighbors.shape[1]
    d = feat.shape[1]
    dbig = er_big.shape[1]
    ch = k
    nch = bw
    nbuf = 6
    look = 4
    assert nch % nbuf == 0
    assert k % 16 == 0
    mesh = plsc.VectorSubcoreMesh(core_axis_name="c", subcore_axis_name="s")
    f32 = jnp.float32
    i32 = jnp.int32

    @functools.partial(
        pl.kernel,
        mesh=mesh,
        out_type=(jax.ShapeDtypeStruct((b, k), f32),
                  jax.ShapeDtypeStruct((b, k), i32),
                  jax.ShapeDtypeStruct((b, d), f32),
                  jax.ShapeDtypeStruct((b * k, d), f32),
                  jax.ShapeDtypeStruct((b * k, dbig), f32)),
        scratch_types=[pltpu.VMEM((bw,), i32),
                       pltpu.VMEM((bw, k), i32),
                       pltpu.VMEM((bw, k), i32),
                       pltpu.VMEM((bw, k), i32),
                       pltpu.VMEM((bw, k), f32),
                       pltpu.VMEM((bw, d), f32)]
                      + [pltpu.VMEM((ch, d), f32)] * nbuf
                      + [pltpu.VMEM((ch, dbig), f32)] * nbuf
                      + [pltpu.SemaphoreType.DMA] * (2 * nbuf + 1),
        compiler_params=pltpu.CompilerParams(use_tc_tiling_on_sc=False),
    )
    def gather(nodes_h, ngh_h, nghe_h, nght_h, feat_h, er_h,
               nght_o, nghe_o, src_o, nf_o, ef_o,
               idx_v, n_v, e_v, es_v, t_v, s_v, *bufs):
        rows = bufs[0:nbuf]
        erows = bufs[nbuf:2 * nbuf]
        in_sem = bufs[2 * nbuf:3 * nbuf]
        out_sem = bufs[3 * nbuf:4 * nbuf]
        sem0 = bufs[4 * nbuf]
        wid = lax.axis_index("s") * nc + lax.axis_index("c")
        base = wid * bw

        pltpu.sync_copy(nodes_h.at[pl.ds(base, bw)], idx_v)
        c1 = pltpu.async_copy(ngh_h.at[idx_v], n_v, sem0)
        c2 = pltpu.async_copy(nghe_h.at[idx_v], e_v, sem0)
        c3 = pltpu.async_copy(nght_h.at[idx_v], t_v, sem0)
        c4 = pltpu.async_copy(feat_h.at[idx_v], s_v, sem0)
        c1.wait()
        c2.wait()
        c3.wait()
        c4.wait()

        def shift_row(r, carry):
            for j in range(k // 16):
                sl = pl.ds(j * 16, 16)
                es_v[r, sl] = lax.bitwise_and(e_v[r, sl], _SEG_ROWS - 1)
            return carry

        lax.fori_loop(0, bw, shift_row, 0)

        pltpu.sync_copy(t_v, nght_o.at[pl.ds(base, bw)])
        pltpu.sync_copy(e_v, nghe_o.at[pl.ds(base, bw)])
        pltpu.sync_copy(s_v, src_o.at[pl.ds(base, bw)])

        def fire_in(s, c):
            pltpu.async_copy(feat_h.at[n_v.at[c]], rows[s], in_sem[s])
            pltpu.async_copy(er_h.at[es_v.at[c]], erows[s], in_sem[s])

        def wait_in(s):
            pltpu.make_async_copy(feat_h.at[n_v.at[0]], rows[s],
                                  in_sem[s]).wait()
            pltpu.make_async_copy(er_h.at[es_v.at[0]], erows[s],
                                  in_sem[s]).wait()

        def fire_out(s, c):
            off = (base + c) * k
            pltpu.async_copy(rows[s], nf_o.at[pl.ds(off, ch)], out_sem[s])
            pltpu.async_copy(erows[s], ef_o.at[pl.ds(off, ch)], out_sem[s])

        def wait_out(s):
            pltpu.make_async_copy(rows[s], nf_o.at[pl.ds(0, ch)],
                                  out_sem[s]).wait()
            pltpu.make_async_copy(erows[s], ef_o.at[pl.ds(0, ch)],
                                  out_sem[s]).wait()

        for c0 in range(look):
            fire_in(c0, c0)

        def outer(g, carry):
            for s in range(nbuf):
                c = g * nbuf + s
                wait_in(s)
                fire_out(s, c)
                c2 = c + look
                s2 = (s + look) % nbuf

                @pl.when(c2 < nch)
                def _():
                    @pl.when(c2 >= nbuf)
                    def _():
                        wait_out(s2)
                    fire_in(s2, c2)
            return carry

        lax.fori_loop(0, nch // nbuf, outer, 0)
        for s in range(nbuf):
            wait_out(s)

    return gather(nodes, neighbors, ngh_e_tab, ngh_t_tab, feat, er_big)


_INV_2PI = 0.15915494309189535
_PI_HI = 6.28125
_PI_LO = 0.0019353071795864769


def _cos(x):
    f32 = jnp.float32
    kf = jnp.round(x * f32(_INV_2PI))
    r = x - kf * f32(_PI_HI)
    r = r - kf * f32(_PI_LO)
    z = r * r
    p = f32(-1.0 / 87178291200.0)
    for coef in (1.0 / 479001600.0, -1.0 / 3628800.0, 1.0 / 40320.0,
                 -1.0 / 720.0, 1.0 / 24.0, -0.5, 1.0):
        p = p * z + f32(coef)
    return p


def _dense_body(src_ref, nf_ref, ef_ref, nghe_ref, ts_ref, nt_ref,
                tw_ref, tb_ref,
                wqn_ref, wqt_ref, wkn_ref, wkeb_ref, wkt_ref,
                wvn_ref, wveb_ref, wvt_ref, wm1a_ref, wm1b_ref, wm2_ref,
                out_ref):
    bq, d = src_ref.shape
    k = nt_ref.shape[1]
    dh = d // 2
    f32 = jnp.float32

    def mm(a, b):
        return lax.dot_general(a, b, (((1,), (0,)), ((), ())),
                               preferred_element_type=f32)

    src = src_ref[...]
    tw = tw_ref[...]
    tb = tb_ref[...]
    delta = ts_ref[...] - nt_ref[...]
    t_enc = _cos(delta[:, :, None] * tw[None, :, :] + tb[None, :, :])
    t2 = t_enc.reshape(bq * k, d)

    seg = nghe_ref[...] // _SEG_ROWS
    lane = lax.broadcasted_iota(jnp.int32, (1, 1, d), 2) >> 4
    mask = (lane == seg[:, :, None]).astype(f32)
    ef3 = ef_ref[...].reshape(bq, k, d)
    efm = (ef3 * mask).reshape(bq * k, d)

    nf = nf_ref[...]
    kk = mm(nf, wkn_ref[...]) + mm(efm, wkeb_ref[...]) + mm(t2, wkt_ref[...])
    vv = mm(nf, wvn_ref[...]) + mm(efm, wveb_ref[...]) + mm(t2, wvt_ref[...])

    qt = _cos(tb)
    q = mm(src, wqn_ref[...]) + mm(qt, wqt_ref[...])

    k3 = kk.reshape(bq, k, d)
    prod = k3 * q[:, None, :]
    scale = f32(1.0 / np.sqrt(dh))
    s0 = jnp.sum(prod[:, :, :dh], axis=-1) * scale
    s1 = jnp.sum(prod[:, :, dh:], axis=-1) * scale

    def softmax(s):
        m = jnp.max(s, axis=-1, keepdims=True)
        e = jnp.exp(s - m)
        return e / jnp.sum(e, axis=-1, keepdims=True)

    a0 = softmax(s0)
    a1 = softmax(s1)
    v3 = vv.reshape(bq, k, d)
    o0 = jnp.sum(a0[:, :, None] * v3[:, :, :dh], axis=1)
    o1 = jnp.sum(a1[:, :, None] * v3[:, :, dh:], axis=1)
    out = jnp.concatenate([o0, o1], axis=-1)

    hmid = jnp.maximum(mm(out, wm1a_ref[...]) + mm(src, wm1b_ref[...]), 0.0)
    out_ref[...] = mm(hmid, wm2_ref[...])


def _dense(src_feat, nf, ef, nghe, ts2, ngh_t, tw2, tb2,
           wqn, wqt, wkn, wkeb, wkt, wvn, wveb, wvt, wm1a, wm1b, wm2):
    b3, d = src_feat.shape
    k = ngh_t.shape[1]
    bq = 128
    assert b3 % bq == 0
    grid = (b3 // bq,)
    full = lambda shape: pl.BlockSpec(shape, lambda i: tuple(0 for _ in shape))
    return pl.pallas_call(
        _dense_body,
        grid=grid,
        in_specs=[
            pl.BlockSpec((bq, d), lambda i: (i, 0)),
            pl.BlockSpec((bq * k, d), lambda i: (i, 0)),
            pl.BlockSpec((bq * k, d), lambda i: (i, 0)),
            pl.BlockSpec((bq, k), lambda i: (i, 0)),
            pl.BlockSpec((bq, 1), lambda i: (i, 0)),
            pl.BlockSpec((bq, k), lambda i: (i, 0)),
            full((1, d)), full((1, d)),
            full((d, d)), full((d, d)),
            full((d, d)), full((d, d)), full((d, d)),
            full((d, d)), full((d, d)), full((d, d)),
            full((d, d)), full((d, d)), full((d, d)),
        ],
        out_specs=pl.BlockSpec((bq, d), lambda i: (i, 0)),
        out_shape=jax.ShapeDtypeStruct((b3, d), jnp.float32),
    )(src_feat, nf, ef, nghe, ts2, ngh_t, tw2, tb2,
      wqn, wqt, wkn, wkeb, wkt, wvn, wveb, wvt, wm1a, wm1b, wm2)


def kernel(source_nodes, destination_nodes, negative_nodes, edge_times,
           edge_idxs, node_raw_features, edge_raw_features, memory_state,
           neighbors, neighbor_edge_idxs, neighbor_times,
           time_w, time_b, W_q, W_k, W_v, W_m1, W_m2):
    del edge_idxs
    d = node_raw_features.shape[1]
    de = edge_raw_features.shape[1]
    e_cnt = edge_raw_features.shape[0]
    pack = 128 // de
    assert e_cnt % pack == 0

    nodes = jnp.concatenate(
        [source_nodes, destination_nodes, negative_nodes]).astype(jnp.int32)
    ts = jnp.concatenate([edge_times, edge_times, edge_times])
    er_big = _repack_edges(edge_raw_features)

    feat = _combined_feat(memory_state, node_raw_features)

    ngh_t, ngh_e, src_feat, nf, ef = _sc_gather_all(
        nodes, neighbors, neighbor_edge_idxs, neighbor_times, feat, er_big)

    tw2 = time_w.reshape(1, d)
    tb2 = time_b.reshape(1, d)
    wqn, wqt = W_q[:d], W_q[d:]
    wkn, wke, wkt = W_k[:d], W_k[d:d + de], W_k[d + de:]
    wvn, wve, wvt = W_v[:d], W_v[d:d + de], W_v[d + de:]
    wkeb = jnp.tile(wke, (pack, 1))
    wveb = jnp.tile(wve, (pack, 1))
    wm1a, wm1b = W_m1[:d], W_m1[d:]

    return _dense(src_feat, nf, ef, ngh_e, ts.reshape(-1, 1), ngh_t, tw2, tb2,
                  wqn, wqt, wkn, wkeb, wkt, wvn, wveb, wvt, wm1a, wm1b, W_m2)


# speedup vs baseline: 1.4991x; 1.4410x over previous
import functools

import jax
import jax.numpy as jnp
import numpy as np
from jax import lax
from jax.experimental import pallas as pl
from jax.experimental.pallas import tpu as pltpu
from jax.experimental.pallas import tpu_sc as plsc


def _add_body(m_ref, r_ref, o_ref):
    o_ref[...] = m_ref[...] + r_ref[...]


def _combined_feat(memory_state, node_raw_features):
    n, d = memory_state.shape
    blk = 2000
    assert n % blk == 0
    return pl.pallas_call(
        _add_body,
        grid=(n // blk,),
        in_specs=[pl.BlockSpec((blk, d), lambda i: (i, 0)),
                  pl.BlockSpec((blk, d), lambda i: (i, 0))],
        out_specs=pl.BlockSpec((blk, d), lambda i: (i, 0)),
        out_shape=jax.ShapeDtypeStruct((n, d), jnp.float32),
    )(memory_state, node_raw_features)


_SEG_ROWS = 131072


def _repack_body(e_cnt, blk, *refs):
    eye_ref = refs[-2]
    out_ref = refs[-1]
    de = refs[0].shape[0]
    i = pl.program_id(0)
    acc = None
    for s, r in enumerate(refs[:-2]):
        x = r[...]
        seg_end = (s + 1) * _SEG_ROWS
        if seg_end > e_cnt:
            col = (s * _SEG_ROWS + i * blk
                   + lax.broadcasted_iota(jnp.int32, (1, blk), 1))
            x = x * (col < e_cnt).astype(jnp.float32)
        part = lax.dot_general(x, eye_ref[s * de:(s + 1) * de, :],
                               (((0,), (0,)), ((), ())),
                               preferred_element_type=jnp.float32)
        acc = part if acc is None else acc + part
    out_ref[...] = acc


def _repack_edges(edge_raw):
    e_cnt, de = edge_raw.shape
    pack = 128 // de
    n_seg = -(-e_cnt // _SEG_ROWS)
    assert n_seg <= pack
    blk = 1024
    assert e_cnt % blk == 0
    nblk = _SEG_ROWS // blk
    er_t = edge_raw.T
    in_specs = [pl.BlockSpec((de, blk), (lambda i, s=s: (0, s * nblk + i)))
                for s in range(n_seg)]
    w = pack * de
    in_specs.append(pl.BlockSpec((w, w), lambda i: (0, 0)))
    return pl.pallas_call(
        functools.partial(_repack_body, e_cnt, blk),
        grid=(nblk,),
        in_specs=in_specs,
        out_specs=pl.BlockSpec((blk, w), lambda i: (i, 0)),
        out_shape=jax.ShapeDtypeStruct((_SEG_ROWS, w), jnp.float32),
    )(*([er_t] * n_seg + [jnp.eye(w, dtype=jnp.float32)]))


def _sc_gather_all(nodes, neighbors, ngh_e_tab, ngh_t_tab, feat, er_big):
    b = nodes.shape[0]
    info = plsc.get_sparse_core_info()
    nc, ns = info.num_cores, info.num_subcores
    nw = nc * ns
    assert b % nw == 0
    bw = b // nw
    k = neighbors.shape[1]
    d = feat.shape[1]
    dbig = er_big.shape[1]
    ch = k
    nch = bw
    nbuf = 6
    look = 4
    assert nch % nbuf == 0
    assert k % 16 == 0
    mesh = plsc.VectorSubcoreMesh(core_axis_name="c", subcore_axis_name="s")
    f32 = jnp.float32
    i32 = jnp.int32

    @functools.partial(
        pl.kernel,
        mesh=mesh,
        out_type=(jax.ShapeDtypeStruct((b, k), f32),
                  jax.ShapeDtypeStruct((b, k), i32),
                  jax.ShapeDtypeStruct((b, d), f32),
                  jax.ShapeDtypeStruct((b * k, d), f32),
                  jax.ShapeDtypeStruct((b * k, dbig), f32)),
        scratch_types=[pltpu.VMEM((bw,), i32),
                       pltpu.VMEM((bw, k), i32),
                       pltpu.VMEM((bw, k), i32),
                       pltpu.VMEM((bw, k), i32),
                       pltpu.VMEM((bw, k), f32),
                       pltpu.VMEM((bw, d), f32)]
                      + [pltpu.VMEM((ch, d), f32)] * nbuf
                      + [pltpu.VMEM((ch, dbig), f32)] * nbuf
                      + [pltpu.SemaphoreType.DMA] * (2 * nbuf + 1),
        compiler_params=pltpu.CompilerParams(use_tc_tiling_on_sc=False),
    )
    def gather(nodes_h, ngh_h, nghe_h, nght_h, feat_h, er_h,
               nght_o, nghe_o, src_o, nf_o, ef_o,
               idx_v, n_v, e_v, es_v, t_v, s_v, *bufs):
        rows = bufs[0:nbuf]
        erows = bufs[nbuf:2 * nbuf]
        in_sem = bufs[2 * nbuf:3 * nbuf]
        out_sem = bufs[3 * nbuf:4 * nbuf]
        sem0 = bufs[4 * nbuf]
        wid = lax.axis_index("s") * nc + lax.axis_index("c")
        base = wid * bw

        pltpu.sync_copy(nodes_h.at[pl.ds(base, bw)], idx_v)
        c1 = pltpu.async_copy(ngh_h.at[idx_v], n_v, sem0)
        c2 = pltpu.async_copy(nghe_h.at[idx_v], e_v, sem0)
        c3 = pltpu.async_copy(nght_h.at[idx_v], t_v, sem0)
        c4 = pltpu.async_copy(feat_h.at[idx_v], s_v, sem0)
        c1.wait()
        c2.wait()
        c3.wait()
        c4.wait()

        def shift_row(r, carry):
            for j in range(k // 16):
                sl = pl.ds(j * 16, 16)
                es_v[r, sl] = lax.bitwise_and(e_v[r, sl], _SEG_ROWS - 1)
            return carry

        lax.fori_loop(0, bw, shift_row, 0)

        pltpu.sync_copy(t_v, nght_o.at[pl.ds(base, bw)])
        pltpu.sync_copy(e_v, nghe_o.at[pl.ds(base, bw)])
        pltpu.sync_copy(s_v, src_o.at[pl.ds(base, bw)])

        def fire_in(s, c):
            pltpu.async_copy(feat_h.at[n_v.at[c]], rows[s], in_sem[s])
            pltpu.async_copy(er_h.at[es_v.at[c]], erows[s], in_sem[s])

        def wait_in(s):
            pltpu.make_async_copy(feat_h.at[n_v.at[0]], rows[s],
                                  in_sem[s]).wait()
            pltpu.make_async_copy(er_h.at[es_v.at[0]], erows[s],
                                  in_sem[s]).wait()

        def fire_out(s, c):
            off = (base + c) * k
            pltpu.async_copy(rows[s], nf_o.at[pl.ds(off, ch)], out_sem[s])
            pltpu.async_copy(erows[s], ef_o.at[pl.ds(off, ch)], out_sem[s])

        def wait_out(s):
            pltpu.make_async_copy(rows[s], nf_o.at[pl.ds(0, ch)],
                                  out_sem[s]).wait()
            pltpu.make_async_copy(erows[s], ef_o.at[pl.ds(0, ch)],
                                  out_sem[s]).wait()

        for c0 in range(look):
            fire_in(c0, c0)

        def outer(g, carry):
            for s in range(nbuf):
                c = g * nbuf + s
                wait_in(s)
                fire_out(s, c)
                c2 = c + look
                s2 = (s + look) % nbuf

                @pl.when(c2 < nch)
                def _():
                    @pl.when(c2 >= nbuf)
                    def _():
                        wait_out(s2)
                    fire_in(s2, c2)
            return carry

        lax.fori_loop(0, nch // nbuf, outer, 0)
        for s in range(nbuf):
            wait_out(s)

    return gather(nodes, neighbors, ngh_e_tab, ngh_t_tab, feat, er_big)


_INV_2PI = 0.15915494309189535
_PI_HI = 6.28125
_PI_LO = 0.0019353071795864769


def _cos(x):
    f32 = jnp.float32
    kf = jnp.round(x * f32(_INV_2PI))
    r = x - kf * f32(_PI_HI)
    r = r - kf * f32(_PI_LO)
    z = r * r
    p = f32(-1.0 / 87178291200.0)
    for coef in (1.0 / 479001600.0, -1.0 / 3628800.0, 1.0 / 40320.0,
                 -1.0 / 720.0, 1.0 / 24.0, -0.5, 1.0):
        p = p * z + f32(coef)
    return p


def _dense_body(src_ref, nf_ref, ef_ref, nghe_ref, ts_ref, nt_ref,
                tw_ref, tb_ref,
                wqn_ref, wqt_ref, wkn_ref, wkeb_ref, wkt_ref,
                wvn_ref, wveb_ref, wvt_ref, wm1a_ref, wm1b_ref, wm2_ref,
                out_ref):
    bq, d = src_ref.shape
    k = nt_ref.shape[1]
    dh = d // 2
    f32 = jnp.float32

    def mm(a, b):
        return lax.dot_general(a, b, (((1,), (0,)), ((), ())),
                               preferred_element_type=f32)

    src = src_ref[...]
    tw = tw_ref[...]
    tb = tb_ref[...]
    delta = ts_ref[...] - nt_ref[...]
    t_enc = _cos(delta[:, :, None] * tw[None, :, :] + tb[None, :, :])
    t2 = t_enc.reshape(bq * k, d)

    seg = nghe_ref[...] // _SEG_ROWS
    lane = lax.broadcasted_iota(jnp.int32, (1, 1, d), 2) >> 4
    mask = (lane == seg[:, :, None]).astype(f32)
    ef3 = ef_ref[...].reshape(bq, k, d)
    efm = (ef3 * mask).reshape(bq * k, d)

    nf = nf_ref[...]
    kk = mm(nf, wkn_ref[...]) + mm(efm, wkeb_ref[...]) + mm(t2, wkt_ref[...])
    vv = mm(nf, wvn_ref[...]) + mm(efm, wveb_ref[...]) + mm(t2, wvt_ref[...])

    qt = _cos(tb)
    q = mm(src, wqn_ref[...]) + mm(qt, wqt_ref[...])

    k3 = kk.reshape(bq, k, d)
    prod = k3 * q[:, None, :]
    scale = f32(1.0 / np.sqrt(dh))
    s0 = jnp.sum(prod[:, :, :dh], axis=-1) * scale
    s1 = jnp.sum(prod[:, :, dh:], axis=-1) * scale

    def softmax(s):
        m = jnp.max(s, axis=-1, keepdims=True)
        e = jnp.exp(s - m)
        return e / jnp.sum(e, axis=-1, keepdims=True)

    a0 = softmax(s0)
    a1 = softmax(s1)
    v3 = vv.reshape(bq, k, d)
    o0 = jnp.sum(a0[:, :, None] * v3[:, :, :dh], axis=1)
    o1 = jnp.sum(a1[:, :, None] * v3[:, :, dh:], axis=1)
    out = jnp.concatenate([o0, o1], axis=-1)

    hmid = jnp.maximum(mm(out, wm1a_ref[...]) + mm(src, wm1b_ref[...]), 0.0)
    out_ref[...] = mm(hmid, wm2_ref[...])


def _dense(src_feat, nf, ef, nghe, ts2, ngh_t, tw2, tb2,
           wqn, wqt, wkn, wkeb, wkt, wvn, wveb, wvt, wm1a, wm1b, wm2):
    b3, d = src_feat.shape
    k = ngh_t.shape[1]
    bq = 128
    assert b3 % bq == 0
    grid = (b3 // bq,)
    full = lambda shape: pl.BlockSpec(shape, lambda i: tuple(0 for _ in shape))
    return pl.pallas_call(
        _dense_body,
        grid=grid,
        in_specs=[
            pl.BlockSpec((bq, d), lambda i: (i, 0)),
            pl.BlockSpec((bq * k, d), lambda i: (i, 0)),
            pl.BlockSpec((bq * k, d), lambda i: (i, 0)),
            pl.BlockSpec((bq, k), lambda i: (i, 0)),
            pl.BlockSpec((bq, 1), lambda i: (i, 0)),
            pl.BlockSpec((bq, k), lambda i: (i, 0)),
            full((1, d)), full((1, d)),
            full((d, d)), full((d, d)),
            full((d, d)), full((d, d)), full((d, d)),
            full((d, d)), full((d, d)), full((d, d)),
            full((d, d)), full((d, d)), full((d, d)),
        ],
        out_specs=pl.BlockSpec((bq, d), lambda i: (i, 0)),
        out_shape=jax.ShapeDtypeStruct((b3, d), jnp.float32),
    )(src_feat, nf, ef, nghe, ts2, ngh_t, tw2, tb2,
      wqn, wqt, wkn, wkeb, wkt, wvn, wveb, wvt, wm1a, wm1b, wm2)


def kernel(source_nodes, destination_nodes, negative_nodes, edge_times,
           edge_idxs, node_raw_features, edge_raw_features, memory_state,
           neighbors, neighbor_edge_idxs, neighbor_times,
           time_w, time_b, W_q, W_k, W_v, W_m1, W_m2):
    del edge_idxs
    d = node_raw_features.shape[1]
    de = edge_raw_features.shape[1]
    e_cnt = edge_raw_features.shape[0]
    pack = 128 // de
    assert e_cnt % pack == 0

    nodes = jnp.concatenate(
        [source_nodes, destination_nodes, negative_nodes]).astype(jnp.int32)
    ts = jnp.concatenate([edge_times, edge_times, edge_times])
    er_big = _repack_edges(edge_raw_features)

    feat = _combined_feat(memory_state, node_raw_features)

    ngh_t, ngh_e, src_feat, nf, ef = _sc_gather_all(
        nodes, neighbors, neighbor_edge_idxs, neighbor_times, feat, er_big)

    tw2 = time_w.reshape(1, d)
    tb2 = time_b.reshape(1, d)
    wqn, wqt = W_q[:d], W_q[d:]
    wkn, wke, wkt = W_k[:d], W_k[d:d + de], W_k[d + de:]
    wvn, wve, wvt = W_v[:d], W_v[d:d + de], W_v[d + de:]
    wkeb = jnp.tile(wke, (pack, 1))
    wveb = jnp.tile(wve, (pack, 1))
    wm1a, wm1b = W_m1[:d], W_m1[d:]

    return _dense(src_feat, nf, ef, ngh_e, ts.reshape(-1, 1), ngh_t, tw2, tb2,
                  wqn, wqt, wkn, wkeb, wkt, wvn, wveb, wvt, wm1a, wm1b, W_m2)


# speedup vs baseline: 1.5021x; 1.0020x over previous
import functools

import jax
import jax.numpy as jnp
import numpy as np
from jax import lax
from jax.experimental import pallas as pl
from jax.experimental.pallas import tpu as pltpu
from jax.experimental.pallas import tpu_sc as plsc


def _add_body(m_ref, r_ref, o_ref):
    o_ref[...] = m_ref[...] + r_ref[...]


def _combined_feat(memory_state, node_raw_features):
    n, d = memory_state.shape
    blk = 2000
    assert n % blk == 0
    return pl.pallas_call(
        _add_body,
        grid=(n // blk,),
        in_specs=[pl.BlockSpec((blk, d), lambda i: (i, 0)),
                  pl.BlockSpec((blk, d), lambda i: (i, 0))],
        out_specs=pl.BlockSpec((blk, d), lambda i: (i, 0)),
        out_shape=jax.ShapeDtypeStruct((n, d), jnp.float32),
    )(memory_state, node_raw_features)


_SEG_ROWS = 131072


def _repack_body(e_cnt, blk, *refs):
    eye_ref = refs[-2]
    out_ref = refs[-1]
    de = refs[0].shape[0]
    i = pl.program_id(0)
    acc = None
    for s, r in enumerate(refs[:-2]):
        x = r[...]
        seg_end = (s + 1) * _SEG_ROWS
        if seg_end > e_cnt:
            col = (s * _SEG_ROWS + i * blk
                   + lax.broadcasted_iota(jnp.int32, (de, blk), 1))
            x = jnp.where(col < e_cnt, x, 0.0)
        part = lax.dot_general(x, eye_ref[s * de:(s + 1) * de, :],
                               (((0,), (0,)), ((), ())),
                               preferred_element_type=jnp.float32)
        acc = part if acc is None else acc + part
    out_ref[...] = acc


def _repack_edges(edge_raw):
    e_cnt, de = edge_raw.shape
    pack = 128 // de
    n_seg = -(-e_cnt // _SEG_ROWS)
    assert n_seg <= pack
    blk = 1024
    assert e_cnt % blk == 0
    nblk = _SEG_ROWS // blk
    er_t = edge_raw.T
    in_specs = [pl.BlockSpec((de, blk), (lambda i, s=s: (0, s * nblk + i)))
                for s in range(n_seg)]
    w = pack * de
    in_specs.append(pl.BlockSpec((w, w), lambda i: (0, 0)))
    return pl.pallas_call(
        functools.partial(_repack_body, e_cnt, blk),
        grid=(nblk,),
        in_specs=in_specs,
        out_specs=pl.BlockSpec((blk, w), lambda i: (i, 0)),
        out_shape=jax.ShapeDtypeStruct((_SEG_ROWS, w), jnp.float32),
    )(*([er_t] * n_seg + [jnp.eye(w, dtype=jnp.float32)]))


def _sc_gather_all(nodes, neighbors, ngh_e_tab, ngh_t_tab, feat, er_big):
    b = nodes.shape[0]
    info = plsc.get_sparse_core_info()
    nc, ns = info.num_cores, info.num_subcores
    nw = nc * ns
    assert b % nw == 0
    bw = b // nw
    k = neighbors.shape[1]
    d = feat.shape[1]
    dbig = er_big.shape[1]
    ch = k
    nch = bw
    nbuf = 6
    look = 4
    assert nch % nbuf == 0
    assert k % 16 == 0
    mesh = plsc.VectorSubcoreMesh(core_axis_name="c", subcore_axis_name="s")
    f32 = jnp.float32
    i32 = jnp.int32

    @functools.partial(
        pl.kernel,
        mesh=mesh,
        out_type=(jax.ShapeDtypeStruct((b, k), f32),
                  jax.ShapeDtypeStruct((b, k), i32),
                  jax.ShapeDtypeStruct((b, d), f32),
                  jax.ShapeDtypeStruct((b * k, d), f32),
                  jax.ShapeDtypeStruct((b * k, dbig), f32)),
        scratch_types=[pltpu.VMEM((bw,), i32),
                       pltpu.VMEM((bw, k), i32),
                       pltpu.VMEM((bw, k), i32),
                       pltpu.VMEM((bw, k), i32),
                       pltpu.VMEM((bw, k), f32),
                       pltpu.VMEM((bw, d), f32)]
                      + [pltpu.VMEM((ch, d), f32)] * nbuf
                      + [pltpu.VMEM((ch, dbig), f32)] * nbuf
                      + [pltpu.SemaphoreType.DMA] * (2 * nbuf + 1),
        compiler_params=pltpu.CompilerParams(use_tc_tiling_on_sc=False),
    )
    def gather(nodes_h, ngh_h, nghe_h, nght_h, feat_h, er_h,
               nght_o, nghe_o, src_o, nf_o, ef_o,
               idx_v, n_v, e_v, es_v, t_v, s_v, *bufs):
        rows = bufs[0:nbuf]
        erows = bufs[nbuf:2 * nbuf]
        in_sem = bufs[2 * nbuf:3 * nbuf]
        out_sem = bufs[3 * nbuf:4 * nbuf]
        sem0 = bufs[4 * nbuf]
        wid = lax.axis_index("s") * nc + lax.axis_index("c")
        base = wid * bw

        pltpu.sync_copy(nodes_h.at[pl.ds(base, bw)], idx_v)
        c1 = pltpu.async_copy(ngh_h.at[idx_v], n_v, sem0)
        c2 = pltpu.async_copy(nghe_h.at[idx_v], e_v, sem0)
        c3 = pltpu.async_copy(nght_h.at[idx_v], t_v, sem0)
        c4 = pltpu.async_copy(feat_h.at[idx_v], s_v, sem0)
        c1.wait()
        c2.wait()
        c3.wait()
        c4.wait()

        def shift_row(r, carry):
            for j in range(k // 16):
                sl = pl.ds(j * 16, 16)
                es_v[r, sl] = lax.bitwise_and(e_v[r, sl], _SEG_ROWS - 1)
            return carry

        lax.fori_loop(0, bw, shift_row, 0)

        pltpu.sync_copy(t_v, nght_o.at[pl.ds(base, bw)])
        pltpu.sync_copy(e_v, nghe_o.at[pl.ds(base, bw)])
        pltpu.sync_copy(s_v, src_o.at[pl.ds(base, bw)])

        def fire_in(s, c):
            pltpu.async_copy(feat_h.at[n_v.at[c]], rows[s], in_sem[s])
            pltpu.async_copy(er_h.at[es_v.at[c]], erows[s], in_sem[s])

        def wait_in(s):
            pltpu.make_async_copy(feat_h.at[n_v.at[0]], rows[s],
                                  in_sem[s]).wait()
            pltpu.make_async_copy(er_h.at[es_v.at[0]], erows[s],
                                  in_sem[s]).wait()

        def fire_out(s, c):
            off = (base + c) * k
            pltpu.async_copy(rows[s], nf_o.at[pl.ds(off, ch)], out_sem[s])
            pltpu.async_copy(erows[s], ef_o.at[pl.ds(off, ch)], out_sem[s])

        def wait_out(s):
            pltpu.make_async_copy(rows[s], nf_o.at[pl.ds(0, ch)],
                                  out_sem[s]).wait()
            pltpu.make_async_copy(erows[s], ef_o.at[pl.ds(0, ch)],
                                  out_sem[s]).wait()

        for c0 in range(look):
            fire_in(c0, c0)

        def outer(g, carry):
            for s in range(nbuf):
                c = g * nbuf + s
                wait_in(s)
                fire_out(s, c)
                c2 = c + look
                s2 = (s + look) % nbuf

                @pl.when(c2 < nch)
                def _():
                    @pl.when(c2 >= nbuf)
                    def _():
                        wait_out(s2)
                    fire_in(s2, c2)
            return carry

        lax.fori_loop(0, nch // nbuf, outer, 0)
        for s in range(nbuf):
            wait_out(s)

    return gather(nodes, neighbors, ngh_e_tab, ngh_t_tab, feat, er_big)


_INV_2PI = 0.15915494309189535
_PI_HI = 6.28125
_PI_LO = 0.0019353071795864769


def _cos(x):
    f32 = jnp.float32
    kf = jnp.round(x * f32(_INV_2PI))
    r = x - kf * f32(_PI_HI)
    r = r - kf * f32(_PI_LO)
    z = r * r
    p = f32(-1.0 / 87178291200.0)
    for coef in (1.0 / 479001600.0, -1.0 / 3628800.0, 1.0 / 40320.0,
                 -1.0 / 720.0, 1.0 / 24.0, -0.5, 1.0):
        p = p * z + f32(coef)
    return p


def _dense_body(src_ref, nf_ref, ef_ref, nghe_ref, ts_ref, nt_ref,
                tw_ref, tb_ref,
                wqn_ref, wqt_ref, wkn_ref, wkeb_ref, wkt_ref,
                wvn_ref, wveb_ref, wvt_ref, wm1a_ref, wm1b_ref, wm2_ref,
                out_ref):
    bq, d = src_ref.shape
    k = nt_ref.shape[1]
    dh = d // 2
    f32 = jnp.float32

    def mm(a, b):
        return lax.dot_general(a, b, (((1,), (0,)), ((), ())),
                               preferred_element_type=f32)

    src = src_ref[...]
    tw = tw_ref[...]
    tb = tb_ref[...]
    delta = ts_ref[...] - nt_ref[...]
    t_enc = _cos(delta[:, :, None] * tw[None, :, :] + tb[None, :, :])
    t2 = t_enc.reshape(bq * k, d)

    seg = nghe_ref[...] // _SEG_ROWS
    lane = lax.broadcasted_iota(jnp.int32, (1, 1, d), 2) >> 4
    ef3 = ef_ref[...].reshape(bq, k, d)
    efm = jnp.where(lane == seg[:, :, None], ef3, 0.0).reshape(bq * k, d)

    nf = nf_ref[...]
    kk = mm(nf, wkn_ref[...]) + mm(efm, wkeb_ref[...]) + mm(t2, wkt_ref[...])
    vv = mm(nf, wvn_ref[...]) + mm(efm, wveb_ref[...]) + mm(t2, wvt_ref[...])

    qt = _cos(tb)
    q = mm(src, wqn_ref[...]) + mm(qt, wqt_ref[...])

    k3 = kk.reshape(bq, k, d)
    prod = k3 * q[:, None, :]
    scale = f32(1.0 / np.sqrt(dh))
    s0 = jnp.sum(prod[:, :, :dh], axis=-1) * scale
    s1 = jnp.sum(prod[:, :, dh:], axis=-1) * scale

    def softmax(s):
        m = jnp.max(s, axis=-1, keepdims=True)
        e = jnp.exp(s - m)
        return e / jnp.sum(e, axis=-1, keepdims=True)

    a0 = softmax(s0)
    a1 = softmax(s1)
    v3 = vv.reshape(bq, k, d)
    o0 = jnp.sum(a0[:, :, None] * v3[:, :, :dh], axis=1)
    o1 = jnp.sum(a1[:, :, None] * v3[:, :, dh:], axis=1)
    out = jnp.concatenate([o0, o1], axis=-1)

    hmid = jnp.maximum(mm(out, wm1a_ref[...]) + mm(src, wm1b_ref[...]), 0.0)
    out_ref[...] = mm(hmid, wm2_ref[...])


def _dense(src_feat, nf, ef, nghe, ts2, ngh_t, tw2, tb2,
           wqn, wqt, wkn, wkeb, wkt, wvn, wveb, wvt, wm1a, wm1b, wm2):
    b3, d = src_feat.shape
    k = ngh_t.shape[1]
    bq = 128
    assert b3 % bq == 0
    grid = (b3 // bq,)
    full = lambda shape: pl.BlockSpec(shape, lambda i: tuple(0 for _ in shape))
    return pl.pallas_call(
        _dense_body,
        grid=grid,
        in_specs=[
            pl.BlockSpec((bq, d), lambda i: (i, 0)),
            pl.BlockSpec((bq * k, d), lambda i: (i, 0)),
            pl.BlockSpec((bq * k, d), lambda i: (i, 0)),
            pl.BlockSpec((bq, k), lambda i: (i, 0)),
            pl.BlockSpec((bq, 1), lambda i: (i, 0)),
            pl.BlockSpec((bq, k), lambda i: (i, 0)),
            full((1, d)), full((1, d)),
            full((d, d)), full((d, d)),
            full((d, d)), full((d, d)), full((d, d)),
            full((d, d)), full((d, d)), full((d, d)),
            full((d, d)), full((d, d)), full((d, d)),
        ],
        out_specs=pl.BlockSpec((bq, d), lambda i: (i, 0)),
        out_shape=jax.ShapeDtypeStruct((b3, d), jnp.float32),
    )(src_feat, nf, ef, nghe, ts2, ngh_t, tw2, tb2,
      wqn, wqt, wkn, wkeb, wkt, wvn, wveb, wvt, wm1a, wm1b, wm2)


def kernel(source_nodes, destination_nodes, negative_nodes, edge_times,
           edge_idxs, node_raw_features, edge_raw_features, memory_state,
           neighbors, neighbor_edge_idxs, neighbor_times,
           time_w, time_b, W_q, W_k, W_v, W_m1, W_m2):
    del edge_idxs
    d = node_raw_features.shape[1]
    de = edge_raw_features.shape[1]
    e_cnt = edge_raw_features.shape[0]
    pack = 128 // de
    assert e_cnt % pack == 0

    nodes = jnp.concatenate(
        [source_nodes, destination_nodes, negative_nodes]).astype(jnp.int32)
    ts = jnp.concatenate([edge_times, edge_times, edge_times])
    er_big = _repack_edges(edge_raw_features)

    feat = _combined_feat(memory_state, node_raw_features)

    ngh_t, ngh_e, src_feat, nf, ef = _sc_gather_all(
        nodes, neighbors, neighbor_edge_idxs, neighbor_times, feat, er_big)

    tw2 = time_w.reshape(1, d)
    tb2 = time_b.reshape(1, d)
    wqn, wqt = W_q[:d], W_q[d:]
    wkn, wke, wkt = W_k[:d], W_k[d:d + de], W_k[d + de:]
    wvn, wve, wvt = W_v[:d], W_v[d:d + de], W_v[d + de:]
    wkeb = jnp.tile(wke, (pack, 1))
    wveb = jnp.tile(wve, (pack, 1))
    wm1a, wm1b = W_m1[:d], W_m1[d:]

    return _dense(src_feat, nf, ef, ngh_e, ts.reshape(-1, 1), ngh_t, tw2, tb2,
                  wqn, wqt, wkn, wkeb, wkt, wvn, wveb, wvt, wm1a, wm1b, W_m2)
